# baseline (speedup 1.0000x reference)
import jax
import jax.numpy as jnp
from jax.experimental import pallas as pl

FPS_POINTS = 512
NEIGHBORS = 32


def _fps(xyz, npoint, farthest0):
    Bb, Nn, _ = xyz.shape
    batch_idx = jnp.arange(Bb)

    def body(i, state):
        centroids, distance, farthest = state
        centroids = centroids.at[:, i].set(farthest)
        centroid = xyz[batch_idx, farthest, :].reshape(Bb, 1, 3)
        dist = jnp.sum((xyz - centroid) ** 2, -1)
        distance = jnp.minimum(distance, dist)
        farthest = jnp.argmax(distance, -1).astype(jnp.int32)
        return (centroids, distance, farthest)

    centroids0 = jnp.zeros((Bb, npoint), dtype=jnp.int32)
    distance0 = jnp.ones((Bb, Nn), dtype=jnp.float32) * 1e10
    centroids, _, _ = jax.lax.fori_loop(
        0, npoint, body, (centroids0, distance0, farthest0.astype(jnp.int32)))
    return centroids


def _index_points(points, idx):
    Bb = points.shape[0]
    batch = jnp.arange(Bb).reshape((Bb,) + (1,) * (idx.ndim - 1))
    return points[batch, idx]


def _copy_kernel(x_ref, o_ref):
    o_ref[...] = x_ref[...]


def kernel(xyz, x, farthest0):
    fps_idx = _fps(xyz, FPS_POINTS, farthest0)
    new_xyz = _index_points(xyz, fps_idx)
    src, dst = new_xyz, xyz
    dists = -2.0 * jnp.matmul(src, jnp.swapaxes(dst, 1, 2))
    dists = dists + jnp.sum(src ** 2, -1)[..., None]
    dists = dists + jnp.sum(dst ** 2, -1)[:, None, :]
    _, knn_idx = jax.lax.top_k(-dists, NEIGHBORS)
    grouped_xyz = _index_points(xyz, knn_idx) - new_xyz[:, :, None, :]
    grouped_points = _index_points(x, knn_idx)
    out = jnp.concatenate([grouped_xyz, grouped_points], axis=-1)
    out = pl.pallas_call(
        _copy_kernel,
        grid=(out.shape[0], 8),
        in_specs=[pl.BlockSpec((1, out.shape[1] // 8) + out.shape[2:],
                               lambda b, s: (b, s, 0, 0))],
        out_specs=pl.BlockSpec((1, out.shape[1] // 8) + out.shape[2:],
                               lambda b, s: (b, s, 0, 0)),
        out_shape=jax.ShapeDtypeStruct(out.shape, out.dtype),
    )(out)
    return out


# speedup vs baseline: 1.6647x; 1.6647x over previous
import functools

import jax
import jax.numpy as jnp
from jax.experimental import pallas as pl

FPS_POINTS = 512
NEIGHBORS = 32
B, N, C = 8, 4096, 128


def _fps_body(far0_ref, xyz_ref, nxt_ref):
    x0 = xyz_ref[0]
    x1 = xyz_ref[1]
    x2 = xyz_ref[2]
    lane = jax.lax.broadcasted_iota(jnp.int32, (B, N), 1)
    lane_s = jax.lax.broadcasted_iota(jnp.int32, (B, FPS_POINTS), 1)

    def body(i, carry):
        distance, far, a0, a1, a2 = carry
        onehot = (lane == far).astype(jnp.float32)
        c0 = jnp.sum(x0 * onehot, axis=1, keepdims=True)
        c1 = jnp.sum(x1 * onehot, axis=1, keepdims=True)
        c2 = jnp.sum(x2 * onehot, axis=1, keepdims=True)
        sel = lane_s == i
        a0 = jnp.where(sel, c0, a0)
        a1 = jnp.where(sel, c1, a1)
        a2 = jnp.where(sel, c2, a2)
        d = (x0 - c0) ** 2 + (x1 - c1) ** 2 + (x2 - c2) ** 2
        distance = jnp.minimum(distance, d)
        m = jnp.max(distance, axis=1, keepdims=True)
        far = jnp.min(jnp.where(distance == m, lane, N), axis=1, keepdims=True)
        return distance, far, a0, a1, a2

    dist0 = jnp.full((B, N), 1e10, dtype=jnp.float32)
    zeros = jnp.zeros((B, FPS_POINTS), dtype=jnp.float32)
    _, _, a0, a1, a2 = jax.lax.fori_loop(
        0, FPS_POINTS, body, (dist0, far0_ref[...], zeros, zeros, zeros))
    nxt_ref[0] = a0
    nxt_ref[1] = a1
    nxt_ref[2] = a2


def _fps_new_xyz(xyz, farthest0):
    xyz_t = jnp.transpose(xyz, (2, 0, 1))
    far0 = farthest0.astype(jnp.int32).reshape(B, 1)
    return pl.pallas_call(
        _fps_body,
        out_shape=jax.ShapeDtypeStruct((3, B, FPS_POINTS), jnp.float32),
    )(far0, xyz_t)


def kernel(xyz, x, farthest0):
    nxt = _fps_new_xyz(xyz, farthest0)
    new_xyz = jnp.transpose(nxt, (1, 2, 0))
    src, dst = new_xyz, xyz
    dists = -2.0 * jnp.matmul(src, jnp.swapaxes(dst, 1, 2))
    dists = dists + jnp.sum(src ** 2, -1)[..., None]
    dists = dists + jnp.sum(dst ** 2, -1)[:, None, :]
    _, knn_idx = jax.lax.top_k(-dists, NEIGHBORS)
    batch = jnp.arange(B).reshape(B, 1, 1)
    grouped_xyz = xyz[batch, knn_idx] - new_xyz[:, :, None, :]
    grouped_points = x[batch, knn_idx]
    return jnp.concatenate([grouped_xyz, grouped_points], axis=-1)


# speedup vs baseline: 3.3623x; 2.0197x over previous
import functools

import jax
import jax.numpy as jnp
from jax.experimental import pallas as pl
from jax.experimental.pallas import tpu as pltpu
from jax.experimental.pallas import tpu_sc as plsc

FPS_POINTS = 512
NEIGHBORS = 32
B, N, C = 8, 4096, 128

_NC, _NS = 2, 16
_NW = _NC * _NS
_ROWS = B * FPS_POINTS * NEIGHBORS
_ROWS_W = _ROWS // _NW
_CHUNK = 128
_NCHUNK = _ROWS_W // _CHUNK
_QPC = _CHUNK // NEIGHBORS


def _fps_body(far0_ref, xyz_ref, nxt_ref):
    x0 = xyz_ref[0]
    x1 = xyz_ref[1]
    x2 = xyz_ref[2]
    lane = jax.lax.broadcasted_iota(jnp.int32, (B, N), 1)
    lane_s = jax.lax.broadcasted_iota(jnp.int32, (B, FPS_POINTS), 1)

    def body(i, carry):
        distance, far, a0, a1, a2 = carry
        onehot = (lane == far).astype(jnp.float32)
        c0 = jnp.sum(x0 * onehot, axis=1, keepdims=True)
        c1 = jnp.sum(x1 * onehot, axis=1, keepdims=True)
        c2 = jnp.sum(x2 * onehot, axis=1, keepdims=True)
        sel = lane_s == i
        a0 = jnp.where(sel, c0, a0)
        a1 = jnp.where(sel, c1, a1)
        a2 = jnp.where(sel, c2, a2)
        d = (x0 - c0) ** 2 + (x1 - c1) ** 2 + (x2 - c2) ** 2
        distance = jnp.minimum(distance, d)
        m = jnp.max(distance, axis=1, keepdims=True)
        far = jnp.min(jnp.where(distance == m, lane, N), axis=1, keepdims=True)
        return distance, far, a0, a1, a2

    dist0 = jnp.full((B, N), 1e10, dtype=jnp.float32)
    zeros = jnp.zeros((B, FPS_POINTS), dtype=jnp.float32)
    _, _, a0, a1, a2 = jax.lax.fori_loop(
        0, FPS_POINTS, body, (dist0, far0_ref[...], zeros, zeros, zeros))
    nxt_ref[0] = a0
    nxt_ref[1] = a1
    nxt_ref[2] = a2


def _fps_new_xyz(xyz, farthest0):
    xyz_t = jnp.transpose(xyz, (2, 0, 1))
    far0 = farthest0.astype(jnp.int32).reshape(B, 1)
    return pl.pallas_call(
        _fps_body,
        out_shape=jax.ShapeDtypeStruct((3, B, FPS_POINTS), jnp.float32),
    )(far0, xyz_t)


def _sc_gather_body(idx_hbm, xf_hbm, xyzp_hbm, qp_hbm, gp_hbm, gx_hbm,
                    idx_v, xrows_v, xyzrows_v, q_v, gxc_v, sem1, sem2):
    cid = jax.lax.axis_index("c")
    sid = jax.lax.axis_index("s")
    wid = sid * _NC + cid
    base = wid * _ROWS_W
    qwbase = wid * (_ROWS_W // NEIGHBORS)

    def pair_body(ci2, carry):
        pltpu.sync_copy(qp_hbm.at[pl.ds(qwbase + ci2 * 8, 8)], q_v)
        for half in range(2):
            ci = ci2 * 2 + half
            rbase = base + ci * _CHUNK
            pltpu.sync_copy(idx_hbm.at[pl.ds(rbase, _CHUNK)], idx_v)
            cp1 = pltpu.async_copy(xf_hbm.at[idx_v], xrows_v, sem1)
            cp2 = pltpu.async_copy(xyzp_hbm.at[idx_v], xyzrows_v, sem2)
            cp2.wait()
            for j in range(_QPC):
                qvec = q_v[half * _QPC + j, :16]
                for r in range(NEIGHBORS):
                    row = j * NEIGHBORS + r
                    gxc_v[row] = xyzrows_v[row, :16] - qvec
            pltpu.sync_copy(gxc_v, gx_hbm.at[pl.ds(rbase, _CHUNK)])
            cp1.wait()
            pltpu.sync_copy(xrows_v, gp_hbm.at[pl.ds(rbase, _CHUNK)])
        return carry

    jax.lax.fori_loop(0, _NCHUNK // 2, pair_body, 0)


def _sc_gather(idx_flat, xf, xyzp, qp):
    mesh = plsc.VectorSubcoreMesh(core_axis_name="c", subcore_axis_name="s")
    return pl.kernel(
        _sc_gather_body,
        out_type=(jax.ShapeDtypeStruct((_ROWS, C), jnp.float32),
                  jax.ShapeDtypeStruct((_ROWS, 16), jnp.float32)),
        mesh=mesh,
        scratch_types=[
            pltpu.VMEM((_CHUNK,), jnp.int32),
            pltpu.VMEM((_CHUNK, C), jnp.float32),
            pltpu.VMEM((_CHUNK, 128), jnp.float32),
            pltpu.VMEM((8, 128), jnp.float32),
            pltpu.VMEM((_CHUNK, 16), jnp.float32),
            pltpu.SemaphoreType.DMA,
            pltpu.SemaphoreType.DMA,
        ],
    )(idx_flat, xf, xyzp, qp)


def kernel(xyz, x, farthest0):
    nxt = _fps_new_xyz(xyz, farthest0)
    new_xyz = jnp.transpose(nxt, (1, 2, 0))
    src, dst = new_xyz, xyz
    dists = -2.0 * jnp.matmul(src, jnp.swapaxes(dst, 1, 2))
    dists = dists + jnp.sum(src ** 2, -1)[..., None]
    dists = dists + jnp.sum(dst ** 2, -1)[:, None, :]
    _, knn_idx = jax.lax.top_k(-dists, NEIGHBORS)
    idx_flat = (knn_idx + (jnp.arange(B) * N).reshape(B, 1, 1)).reshape(-1)
    idx_flat = idx_flat.astype(jnp.int32)
    xf = x.reshape(B * N, C)
    xyzp = jnp.pad(xyz, ((0, 0), (0, 0), (0, 125))).reshape(B * N, 128)
    qp = jnp.pad(new_xyz, ((0, 0), (0, 0), (0, 125))).reshape(B * FPS_POINTS, 128)
    gp, gx = _sc_gather(idx_flat, xf, xyzp, qp)
    grouped_points = gp.reshape(B, FPS_POINTS, NEIGHBORS, C)
    grouped_xyz = gx.reshape(B, FPS_POINTS, NEIGHBORS, 16)[..., :3]
    return jnp.concatenate([grouped_xyz, grouped_points], axis=-1)


# speedup vs baseline: 9.5394x; 2.8372x over previous
import functools

import jax
import jax.numpy as jnp
from jax.experimental import pallas as pl
from jax.experimental.pallas import tpu as pltpu
from jax.experimental.pallas import tpu_sc as plsc

FPS_POINTS = 512
NEIGHBORS = 32
B, N, C = 8, 4096, 128

_NC, _NS = 2, 16
_NW = _NC * _NS
_ROWS = B * FPS_POINTS * NEIGHBORS
_ROWS_W = _ROWS // _NW
_CHUNK = 128
_NCHUNK = _ROWS_W // _CHUNK
_QPC = _CHUNK // NEIGHBORS


def _fps_body(far0_ref, xyz_ref, nxt_ref):
    x0 = xyz_ref[0]
    x1 = xyz_ref[1]
    x2 = xyz_ref[2]
    lane = jax.lax.broadcasted_iota(jnp.int32, (B, N), 1)
    lane_s = jax.lax.broadcasted_iota(jnp.int32, (B, FPS_POINTS), 1)

    def body(i, carry):
        distance, far, a0, a1, a2 = carry
        onehot = (lane == far).astype(jnp.float32)
        c0 = jnp.sum(x0 * onehot, axis=1, keepdims=True)
        c1 = jnp.sum(x1 * onehot, axis=1, keepdims=True)
        c2 = jnp.sum(x2 * onehot, axis=1, keepdims=True)
        sel = lane_s == i
        a0 = jnp.where(sel, c0, a0)
        a1 = jnp.where(sel, c1, a1)
        a2 = jnp.where(sel, c2, a2)
        d = (x0 - c0) ** 2 + (x1 - c1) ** 2 + (x2 - c2) ** 2
        distance = jnp.minimum(distance, d)
        m = jnp.max(distance, axis=1, keepdims=True)
        far = jnp.min(jnp.where(distance == m, lane, N), axis=1, keepdims=True)
        return distance, far, a0, a1, a2

    dist0 = jnp.full((B, N), 1e10, dtype=jnp.float32)
    zeros = jnp.zeros((B, FPS_POINTS), dtype=jnp.float32)
    _, _, a0, a1, a2 = jax.lax.fori_loop(
        0, FPS_POINTS, body, (dist0, far0_ref[...], zeros, zeros, zeros))
    nxt_ref[0] = a0
    nxt_ref[1] = a1
    nxt_ref[2] = a2


def _fps_new_xyz(xyz, farthest0):
    xyz_t = jnp.transpose(xyz, (2, 0, 1))
    far0 = farthest0.astype(jnp.int32).reshape(B, 1)
    return pl.pallas_call(
        _fps_body,
        out_shape=jax.ShapeDtypeStruct((3, B, FPS_POINTS), jnp.float32),
    )(far0, xyz_t)


def _sc_gather_body(idx_hbm, xf_hbm, xyzp_hbm, qp_hbm, gp_hbm, gx_hbm,
                    idx_v, xrows_v, xyzrows_v, q_v, gxc_v, sem1, sem2):
    cid = jax.lax.axis_index("c")
    sid = jax.lax.axis_index("s")
    wid = sid * _NC + cid
    base = wid * _ROWS_W
    qwbase = wid * (_ROWS_W // NEIGHBORS)

    def pair_body(ci2, carry):
        pltpu.sync_copy(qp_hbm.at[pl.ds(qwbase + ci2 * 8, 8)], q_v)
        for half in range(2):
            ci = ci2 * 2 + half
            rbase = base + ci * _CHUNK
            pltpu.sync_copy(idx_hbm.at[pl.ds(rbase, _CHUNK)], idx_v)
            cp1 = pltpu.async_copy(xf_hbm.at[idx_v], xrows_v, sem1)
            cp2 = pltpu.async_copy(xyzp_hbm.at[idx_v], xyzrows_v, sem2)
            cp2.wait()
            for j in range(_QPC):
                qvec = q_v[half * _QPC + j, :16]
                for r in range(NEIGHBORS):
                    row = j * NEIGHBORS + r
                    gxc_v[row] = xyzrows_v[row, :16] - qvec
            pltpu.sync_copy(gxc_v, gx_hbm.at[pl.ds(rbase, _CHUNK)])
            cp1.wait()
            pltpu.sync_copy(xrows_v, gp_hbm.at[pl.ds(rbase, _CHUNK)])
        return carry

    jax.lax.fori_loop(0, _NCHUNK // 2, pair_body, 0)


def _sc_gather(idx_flat, xf, xyzp, qp):
    mesh = plsc.VectorSubcoreMesh(core_axis_name="c", subcore_axis_name="s")
    return pl.kernel(
        _sc_gather_body,
        out_type=(jax.ShapeDtypeStruct((_ROWS, C), jnp.float32),
                  jax.ShapeDtypeStruct((_ROWS, 16), jnp.float32)),
        mesh=mesh,
        scratch_types=[
            pltpu.VMEM((_CHUNK,), jnp.int32),
            pltpu.VMEM((_CHUNK, C), jnp.float32),
            pltpu.VMEM((_CHUNK, 128), jnp.float32),
            pltpu.VMEM((8, 128), jnp.float32),
            pltpu.VMEM((_CHUNK, 16), jnp.float32),
            pltpu.SemaphoreType.DMA,
            pltpu.SemaphoreType.DMA,
        ],
    )(idx_flat, xf, xyzp, qp)


def _knn_body(xyz_ref, q_ref, out_ref):
    b = pl.program_id(0)
    xs = xyz_ref[0]
    q = q_ref[0]
    xs2 = xs * xs
    xn = xs2[0:1] + xs2[1:2] + xs2[2:3]
    qn = jnp.sum(q * q, axis=1, keepdims=True)
    g = jax.lax.dot_general(q, xs, (((1,), (0,)), ((), ())),
                            precision=jax.lax.Precision.DEFAULT)
    d = (-2.0 * g + qn) + xn
    lane = jax.lax.broadcasted_iota(jnp.int32, (FPS_POINTS, N), 1)
    lane_k = jax.lax.broadcasted_iota(jnp.int32, (FPS_POINTS, NEIGHBORS), 1)
    big = jnp.float32(jnp.inf)

    def sel(k, carry):
        d, knn = carry
        m = jnp.min(d, axis=1, keepdims=True)
        idx = jnp.min(jnp.where(d == m, lane, N), axis=1, keepdims=True)
        knn = jnp.where(lane_k == k, idx, knn)
        d = jnp.where(lane == idx, big, d)
        return d, knn

    knn0 = jnp.zeros((FPS_POINTS, NEIGHBORS), dtype=jnp.int32)
    _, knn = jax.lax.fori_loop(0, NEIGHBORS, sel, (d, knn0))
    out_ref[0] = knn + b * N


def _knn_flat_idx(xyz, new_xyz):
    xyz_t = jnp.transpose(xyz, (0, 2, 1))
    return pl.pallas_call(
        _knn_body,
        grid=(B,),
        in_specs=[
            pl.BlockSpec((1, 3, N), lambda b: (b, 0, 0)),
            pl.BlockSpec((1, FPS_POINTS, 3), lambda b: (b, 0, 0)),
        ],
        out_specs=pl.BlockSpec((1, FPS_POINTS, NEIGHBORS), lambda b: (b, 0, 0)),
        out_shape=jax.ShapeDtypeStruct((B, FPS_POINTS, NEIGHBORS), jnp.int32),
    )(xyz_t, new_xyz)


def kernel(xyz, x, farthest0):
    nxt = _fps_new_xyz(xyz, farthest0)
    new_xyz = jnp.transpose(nxt, (1, 2, 0))
    idx_flat = _knn_flat_idx(xyz, new_xyz).reshape(-1)
    xf = x.reshape(B * N, C)
    xyzp = jnp.pad(xyz, ((0, 0), (0, 0), (0, 125))).reshape(B * N, 128)
    qp = jnp.pad(new_xyz, ((0, 0), (0, 0), (0, 125))).reshape(B * FPS_POINTS, 128)
    gp, gx = _sc_gather(idx_flat, xf, xyzp, qp)
    grouped_points = gp.reshape(B, FPS_POINTS, NEIGHBORS, C)
    grouped_xyz = gx.reshape(B, FPS_POINTS, NEIGHBORS, 16)[..., :3]
    return jnp.concatenate([grouped_xyz, grouped_points], axis=-1)


# speedup vs baseline: 9.7466x; 1.0217x over previous
import functools

import jax
import jax.numpy as jnp
from jax.experimental import pallas as pl
from jax.experimental.pallas import tpu as pltpu
from jax.experimental.pallas import tpu_sc as plsc

FPS_POINTS = 512
NEIGHBORS = 32
B, N, C = 8, 4096, 128

_NC, _NS = 2, 16
_NW = _NC * _NS
_ROWS = B * FPS_POINTS * NEIGHBORS
_ROWS_W = _ROWS // _NW
_CHUNK = 128
_NCHUNK = _ROWS_W // _CHUNK
_QPC = _CHUNK // NEIGHBORS


def _fps_body(far0_ref, xyz_ref, nxt_ref):
    x0 = xyz_ref[0]
    x1 = xyz_ref[1]
    x2 = xyz_ref[2]
    lane = jax.lax.broadcasted_iota(jnp.int32, (B, N), 1)
    lane_s = jax.lax.broadcasted_iota(jnp.int32, (B, FPS_POINTS), 1)

    def body(i, carry):
        distance, far, a0, a1, a2 = carry
        onehot = (lane == far).astype(jnp.float32)
        c0 = jnp.sum(x0 * onehot, axis=1, keepdims=True)
        c1 = jnp.sum(x1 * onehot, axis=1, keepdims=True)
        c2 = jnp.sum(x2 * onehot, axis=1, keepdims=True)
        sel = lane_s == i
        a0 = jnp.where(sel, c0, a0)
        a1 = jnp.where(sel, c1, a1)
        a2 = jnp.where(sel, c2, a2)
        d = (x0 - c0) ** 2 + (x1 - c1) ** 2 + (x2 - c2) ** 2
        distance = jnp.minimum(distance, d)
        far = jnp.argmax(distance, axis=1).astype(jnp.int32).reshape(B, 1)
        return distance, far, a0, a1, a2

    dist0 = jnp.full((B, N), 1e10, dtype=jnp.float32)
    zeros = jnp.zeros((B, FPS_POINTS), dtype=jnp.float32)
    _, _, a0, a1, a2 = jax.lax.fori_loop(
        0, FPS_POINTS, body, (dist0, far0_ref[...], zeros, zeros, zeros))
    nxt_ref[0] = a0
    nxt_ref[1] = a1
    nxt_ref[2] = a2


def _fps_new_xyz(xyz, farthest0):
    xyz_t = jnp.transpose(xyz, (2, 0, 1))
    far0 = farthest0.astype(jnp.int32).reshape(B, 1)
    return pl.pallas_call(
        _fps_body,
        out_shape=jax.ShapeDtypeStruct((3, B, FPS_POINTS), jnp.float32),
    )(far0, xyz_t)


def _sc_gather_body(idx_hbm, xf_hbm, xyzp_hbm, qp_hbm, gp_hbm, gx_hbm,
                    idx_v, xrows_v, xyzrows_v, q_v, gxc_v, sem1, sem2):
    cid = jax.lax.axis_index("c")
    sid = jax.lax.axis_index("s")
    wid = sid * _NC + cid
    base = wid * _ROWS_W
    qwbase = wid * (_ROWS_W // NEIGHBORS)

    def pair_body(ci2, carry):
        pltpu.sync_copy(qp_hbm.at[pl.ds(qwbase + ci2 * 8, 8)], q_v)
        for half in range(2):
            ci = ci2 * 2 + half
            rbase = base + ci * _CHUNK
            pltpu.sync_copy(idx_hbm.at[pl.ds(rbase, _CHUNK)], idx_v)
            cp1 = pltpu.async_copy(xf_hbm.at[idx_v], xrows_v, sem1)
            cp2 = pltpu.async_copy(xyzp_hbm.at[idx_v], xyzrows_v, sem2)
            cp2.wait()
            for j in range(_QPC):
                qvec = q_v[half * _QPC + j, :16]
                for r in range(NEIGHBORS):
                    row = j * NEIGHBORS + r
                    gxc_v[row] = xyzrows_v[row, :16] - qvec
            pltpu.sync_copy(gxc_v, gx_hbm.at[pl.ds(rbase, _CHUNK)])
            cp1.wait()
            pltpu.sync_copy(xrows_v, gp_hbm.at[pl.ds(rbase, _CHUNK)])
        return carry

    jax.lax.fori_loop(0, _NCHUNK // 2, pair_body, 0)


def _sc_gather(idx_flat, xf, xyzp, qp):
    mesh = plsc.VectorSubcoreMesh(core_axis_name="c", subcore_axis_name="s")
    return pl.kernel(
        _sc_gather_body,
        out_type=(jax.ShapeDtypeStruct((_ROWS, C), jnp.float32),
                  jax.ShapeDtypeStruct((_ROWS, 16), jnp.float32)),
        mesh=mesh,
        scratch_types=[
            pltpu.VMEM((_CHUNK,), jnp.int32),
            pltpu.VMEM((_CHUNK, C), jnp.float32),
            pltpu.VMEM((_CHUNK, 128), jnp.float32),
            pltpu.VMEM((8, 128), jnp.float32),
            pltpu.VMEM((_CHUNK, 16), jnp.float32),
            pltpu.SemaphoreType.DMA,
            pltpu.SemaphoreType.DMA,
        ],
    )(idx_flat, xf, xyzp, qp)


def _knn_body(xyz_ref, q_ref, out_ref):
    b = pl.program_id(0)
    xs = xyz_ref[0]
    q = q_ref[0]
    xs2 = xs * xs
    xn = xs2[0:1] + xs2[1:2] + xs2[2:3]
    qn = jnp.sum(q * q, axis=1, keepdims=True)
    g = jax.lax.dot_general(q, xs, (((1,), (0,)), ((), ())),
                            precision=jax.lax.Precision.DEFAULT)
    d = (-2.0 * g + qn) + xn
    lane = jax.lax.broadcasted_iota(jnp.int32, (FPS_POINTS, N), 1)
    lane_k = jax.lax.broadcasted_iota(jnp.int32, (FPS_POINTS, NEIGHBORS), 1)
    big = jnp.float32(jnp.inf)

    def sel(k, carry):
        d, knn = carry
        idx = jnp.argmin(d, axis=1).astype(jnp.int32).reshape(FPS_POINTS, 1)
        knn = jnp.where(lane_k == k, idx, knn)
        d = jnp.where(lane == idx, big, d)
        return d, knn

    knn0 = jnp.zeros((FPS_POINTS, NEIGHBORS), dtype=jnp.int32)
    _, knn = jax.lax.fori_loop(0, NEIGHBORS, sel, (d, knn0))
    out_ref[0] = knn + b * N


def _knn_flat_idx(xyz, new_xyz):
    xyz_t = jnp.transpose(xyz, (0, 2, 1))
    return pl.pallas_call(
        _knn_body,
        grid=(B,),
        in_specs=[
            pl.BlockSpec((1, 3, N), lambda b: (b, 0, 0)),
            pl.BlockSpec((1, FPS_POINTS, 3), lambda b: (b, 0, 0)),
        ],
        out_specs=pl.BlockSpec((1, FPS_POINTS, NEIGHBORS), lambda b: (b, 0, 0)),
        out_shape=jax.ShapeDtypeStruct((B, FPS_POINTS, NEIGHBORS), jnp.int32),
    )(xyz_t, new_xyz)


def kernel(xyz, x, farthest0):
    nxt = _fps_new_xyz(xyz, farthest0)
    new_xyz = jnp.transpose(nxt, (1, 2, 0))
    idx_flat = _knn_flat_idx(xyz, new_xyz).reshape(-1)
    xf = x.reshape(B * N, C)
    xyzp = jnp.pad(xyz, ((0, 0), (0, 0), (0, 125))).reshape(B * N, 128)
    qp = jnp.pad(new_xyz, ((0, 0), (0, 0), (0, 125))).reshape(B * FPS_POINTS, 128)
    gp, gx = _sc_gather(idx_flat, xf, xyzp, qp)
    grouped_points = gp.reshape(B, FPS_POINTS, NEIGHBORS, C)
    grouped_xyz = gx.reshape(B, FPS_POINTS, NEIGHBORS, 16)[..., :3]
    return jnp.concatenate([grouped_xyz, grouped_points], axis=-1)
